# speedup vs baseline: 18.2935x; 18.2935x over previous
import jax
import jax.numpy as jnp
from jax.experimental import pallas as pl
from jax.experimental.pallas import tpu as pltpu

_B = 16384
_D_IN = 128
_D_EMB = 4096
_K = 32
_BR = 512


def _topk_sae_block(x_ref, w_ref, be_ref, bd_ref, out_ref):
    x = x_ref[...]
    mean = jnp.mean(x, axis=-1, keepdims=True)
    cen = x - mean
    var = jnp.sum(cen * cen, axis=-1, keepdims=True) * (1.0 / (_D_IN - 1))
    xn = cen / (jnp.sqrt(var) + 1e-07)
    xn = xn - bd_ref[...]
    e = jnp.dot(xn, w_ref[...], preferred_element_type=jnp.float32)
    e = jnp.maximum(e + be_ref[...], 0.0)

    ei = jax.lax.bitcast_convert_type(e, jnp.int32)
    lo = jnp.zeros((_BR, 1), jnp.int32)
    hi = jnp.max(ei, axis=-1, keepdims=True) + 1
    for _ in range(31):
        mid = lo + (hi - lo) // 2
        cnt = jnp.sum((ei >= mid).astype(jnp.int32), axis=-1, keepdims=True)
        ge = cnt >= _K
        lo = jnp.where(ge, mid, lo)
        hi = jnp.where(ge, hi, mid)
    out_ref[...] = jnp.where(ei >= lo, e, 0.0)


def kernel(x, encoder_w, encoder_b, decoder_b):
    be = encoder_b.reshape(1, _D_EMB)
    bd = decoder_b.reshape(1, _D_IN)
    grid = (_B // _BR,)
    return pl.pallas_call(
        _topk_sae_block,
        grid=grid,
        in_specs=[
            pl.BlockSpec((_BR, _D_IN), lambda i: (i, 0)),
            pl.BlockSpec((_D_IN, _D_EMB), lambda i: (0, 0)),
            pl.BlockSpec((1, _D_EMB), lambda i: (0, 0)),
            pl.BlockSpec((1, _D_IN), lambda i: (0, 0)),
        ],
        out_specs=pl.BlockSpec((_BR, _D_EMB), lambda i: (i, 0)),
        out_shape=jax.ShapeDtypeStruct((_B, _D_EMB), jnp.float32),
        compiler_params=pltpu.CompilerParams(
            dimension_semantics=("arbitrary",),
        ),
    )(x, encoder_w, be, bd)


# speedup vs baseline: 73.9514x; 4.0425x over previous
import jax
import jax.numpy as jnp
from jax.experimental import pallas as pl
from jax.experimental.pallas import tpu as pltpu

_B = 16384
_D_IN = 128
_D_EMB = 4096
_K = 32
_BR = 128


def _bitonic_merge_desc(v):
    n = len(v)
    j = n // 2
    while j >= 1:
        for i in range(n):
            l = i ^ j
            if l > i:
                a, b = v[i], v[l]
                v[i] = jnp.maximum(a, b)
                v[l] = jnp.minimum(a, b)
        j //= 2
    return v


def _bitonic_sort_desc(v):
    n = len(v)
    k = 2
    while k <= n:
        j = k // 2
        while j >= 1:
            for i in range(n):
                l = i ^ j
                if l > i:
                    a, b = v[i], v[l]
                    if (i & k) == 0:
                        v[i], v[l] = jnp.maximum(a, b), jnp.minimum(a, b)
                    else:
                        v[i], v[l] = jnp.minimum(a, b), jnp.maximum(a, b)
            j //= 2
        k *= 2
    return v


def _merge_top(a, b):
    n = len(a)
    c = [jnp.maximum(a[i], b[n - 1 - i]) for i in range(n)]
    return _bitonic_merge_desc(c)


def _topk_sae_tile(xt_ref, wt_ref, be_ref, bd_ref, out_ref):
    xt = xt_ref[...]
    mean = jnp.mean(xt, axis=0, keepdims=True)
    cen = xt - mean
    var = jnp.sum(cen * cen, axis=0, keepdims=True) * (1.0 / (_D_IN - 1))
    xn = cen / (jnp.sqrt(var) + 1e-07)
    xn = xn - bd_ref[...]
    et = jnp.dot(wt_ref[...], xn, preferred_element_type=jnp.float32)
    et = jnp.maximum(et + be_ref[...], 0.0)

    v4 = et.reshape(16, 32, 8, _BR)
    runs = [v4[:, i] for i in range(32)]
    runs = _bitonic_sort_desc(runs)
    while runs[0].shape[0] > 1:
        h = runs[0].shape[0] // 2
        a = [r[:h] for r in runs]
        b = [r[h:] for r in runs]
        runs = _merge_top(a, b)
    runs = [r[0] for r in runs]
    for sh in (4, 2, 1):
        b = [pltpu.roll(r, sh, axis=0) for r in runs]
        runs = _merge_top(runs, b)
    thresh = runs[_K - 1][0:1, :]

    masked = jnp.where(et >= thresh, et, 0.0)
    for c in range(_D_EMB // _BR):
        out_ref[:, c * _BR:(c + 1) * _BR] = masked[c * _BR:(c + 1) * _BR, :].T


def kernel(x, encoder_w, encoder_b, decoder_b):
    xt = x.T
    wt = encoder_w.T
    be = encoder_b.reshape(_D_EMB, 1)
    bd = decoder_b.reshape(_D_IN, 1)
    grid = (_B // _BR,)
    return pl.pallas_call(
        _topk_sae_tile,
        grid=grid,
        in_specs=[
            pl.BlockSpec((_D_IN, _BR), lambda i: (0, i)),
            pl.BlockSpec((_D_EMB, _D_IN), lambda i: (0, 0)),
            pl.BlockSpec((_D_EMB, 1), lambda i: (0, 0)),
            pl.BlockSpec((_D_IN, 1), lambda i: (0, 0)),
        ],
        out_specs=pl.BlockSpec((_BR, _D_EMB), lambda i: (i, 0)),
        out_shape=jax.ShapeDtypeStruct((_B, _D_EMB), jnp.float32),
        compiler_params=pltpu.CompilerParams(
            dimension_semantics=("arbitrary",),
        ),
    )(xt, wt, be, bd)


# speedup vs baseline: 75.1856x; 1.0167x over previous
import jax
import jax.numpy as jnp
from jax.experimental import pallas as pl
from jax.experimental.pallas import tpu as pltpu

_B = 16384
_D_IN = 128
_D_EMB = 4096
_K = 32
_BR = 256


def _bitonic_merge_desc(v):
    n = len(v)
    j = n // 2
    while j >= 1:
        for i in range(n):
            l = i ^ j
            if l > i:
                a, b = v[i], v[l]
                v[i] = jnp.maximum(a, b)
                v[l] = jnp.minimum(a, b)
        j //= 2
    return v


def _bitonic_sort_desc(v):
    n = len(v)
    k = 2
    while k <= n:
        j = k // 2
        while j >= 1:
            for i in range(n):
                l = i ^ j
                if l > i:
                    a, b = v[i], v[l]
                    if (i & k) == 0:
                        v[i], v[l] = jnp.maximum(a, b), jnp.minimum(a, b)
                    else:
                        v[i], v[l] = jnp.minimum(a, b), jnp.maximum(a, b)
            j //= 2
        k *= 2
    return v


def _merge_top(a, b):
    n = len(a)
    c = [jnp.maximum(a[i], b[n - 1 - i]) for i in range(n)]
    return _bitonic_merge_desc(c)


def _topk_sae_tile(xt_ref, wt_ref, be_ref, bd_ref, out_ref):
    xt = xt_ref[...]
    mean = jnp.mean(xt, axis=0, keepdims=True)
    cen = xt - mean
    var = jnp.sum(cen * cen, axis=0, keepdims=True) * (1.0 / (_D_IN - 1))
    xn = cen / (jnp.sqrt(var) + 1e-07)
    xn = xn - bd_ref[...]
    et = jnp.dot(wt_ref[...], xn, preferred_element_type=jnp.float32)
    et = jnp.maximum(et + be_ref[...], 0.0)

    v4 = et.reshape(16, 32, 8, _BR)
    runs = [v4[:, i] for i in range(32)]
    runs = _bitonic_sort_desc(runs)
    while runs[0].shape[0] > 1:
        h = runs[0].shape[0] // 2
        a = [r[:h] for r in runs]
        b = [r[h:] for r in runs]
        runs = _merge_top(a, b)
    runs = [r[0] for r in runs]
    for sh in (4, 2, 1):
        a = [r[:sh] for r in runs]
        b = [r[sh:] for r in runs]
        runs = _merge_top(a, b)
    thresh = runs[_K - 1]

    masked = jnp.where(et >= thresh, et, 0.0)
    for c in range(_D_EMB // _BR):
        out_ref[:, c * _BR:(c + 1) * _BR] = masked[c * _BR:(c + 1) * _BR, :].T


def kernel(x, encoder_w, encoder_b, decoder_b):
    xt = x.T
    wt = encoder_w.T
    be = encoder_b.reshape(_D_EMB, 1)
    bd = decoder_b.reshape(_D_IN, 1)
    grid = (_B // _BR,)
    return pl.pallas_call(
        _topk_sae_tile,
        grid=grid,
        in_specs=[
            pl.BlockSpec((_D_IN, _BR), lambda i: (0, i)),
            pl.BlockSpec((_D_EMB, _D_IN), lambda i: (0, 0)),
            pl.BlockSpec((_D_EMB, 1), lambda i: (0, 0)),
            pl.BlockSpec((_D_IN, 1), lambda i: (0, 0)),
        ],
        out_specs=pl.BlockSpec((_BR, _D_EMB), lambda i: (i, 0)),
        out_shape=jax.ShapeDtypeStruct((_B, _D_EMB), jnp.float32),
        compiler_params=pltpu.CompilerParams(
            dimension_semantics=("arbitrary",),
        ),
    )(xt, wt, be, bd)


# speedup vs baseline: 83.0406x; 1.1045x over previous
import jax
import jax.numpy as jnp
from jax.experimental import pallas as pl
from jax.experimental.pallas import tpu as pltpu

_B = 16384
_D_IN = 128
_D_EMB = 4096
_K = 32
_BR = 256


def _bitonic_merge_desc(v):
    n = len(v)
    j = n // 2
    while j >= 1:
        for i in range(n):
            l = i ^ j
            if l > i:
                a, b = v[i], v[l]
                v[i] = jnp.maximum(a, b)
                v[l] = jnp.minimum(a, b)
        j //= 2
    return v


def _batcher_pairs(n):
    pairs = []

    def oddeven_merge(lo, n2, r):
        step = r * 2
        if step < n2:
            oddeven_merge(lo, n2, step)
            oddeven_merge(lo + r, n2, step)
            for i in range(lo + r, lo + n2 - r, step):
                pairs.append((i, i + r))
        else:
            pairs.append((lo, lo + r))

    def rec(lo, n2):
        if n2 > 1:
            m = n2 // 2
            rec(lo, m)
            rec(lo + m, m)
            oddeven_merge(lo, n2, 1)

    rec(0, n)
    return pairs


def _oddeven_sort_desc(v):
    for i, j in _batcher_pairs(len(v)):
        a, b = v[i], v[j]
        v[i] = jnp.maximum(a, b)
        v[j] = jnp.minimum(a, b)
    return v


def _merge_top(a, b):
    n = len(a)
    c = [jnp.maximum(a[i], b[n - 1 - i]) for i in range(n)]
    return _bitonic_merge_desc(c)


def _topk_sae_tile(x_ref, wt_ref, be_ref, bd_ref, out_ref):
    xb = x_ref[...]
    mean = jnp.mean(xb, axis=1, keepdims=True)
    cen = xb - mean
    var = jnp.sum(cen * cen, axis=1, keepdims=True) * (1.0 / (_D_IN - 1))
    xn = cen / (jnp.sqrt(var) + 1e-07)
    xn = xn - bd_ref[...]
    et = jax.lax.dot_general(wt_ref[...], xn, (((1,), (1,)), ((), ())),
                             preferred_element_type=jnp.float32)
    et = jnp.maximum(et + be_ref[...], 0.0)

    v4 = et.reshape(16, 32, 8, _BR)
    runs = [v4[:, i] for i in range(32)]
    runs = _oddeven_sort_desc(runs)
    while runs[0].shape[0] > 1:
        h = runs[0].shape[0] // 2
        a = [r[:h] for r in runs]
        b = [r[h:] for r in runs]
        runs = _merge_top(a, b)
    runs = [r[0] for r in runs]
    for sh in (4, 2, 1):
        a = [r[:sh] for r in runs]
        b = [r[sh:] for r in runs]
        runs = _merge_top(a, b)
    thresh = runs[_K - 1]

    masked = jnp.where(et >= thresh, et, 0.0)
    for c in range(_D_EMB // _BR):
        out_ref[:, c * _BR:(c + 1) * _BR] = masked[c * _BR:(c + 1) * _BR, :].T


def kernel(x, encoder_w, encoder_b, decoder_b):
    wt = encoder_w.T
    be = encoder_b.reshape(_D_EMB, 1)
    bd = decoder_b.reshape(1, _D_IN)
    grid = (_B // _BR,)
    return pl.pallas_call(
        _topk_sae_tile,
        grid=grid,
        in_specs=[
            pl.BlockSpec((_BR, _D_IN), lambda i: (i, 0)),
            pl.BlockSpec((_D_EMB, _D_IN), lambda i: (0, 0)),
            pl.BlockSpec((_D_EMB, 1), lambda i: (0, 0)),
            pl.BlockSpec((1, _D_IN), lambda i: (0, 0)),
        ],
        out_specs=pl.BlockSpec((_BR, _D_EMB), lambda i: (i, 0)),
        out_shape=jax.ShapeDtypeStruct((_B, _D_EMB), jnp.float32),
        compiler_params=pltpu.CompilerParams(
            dimension_semantics=("arbitrary",),
        ),
    )(x, wt, be, bd)
